# baseline (speedup 1.0000x reference)
import jax
import jax.numpy as jnp
from jax.experimental import pallas as pl
from jax.experimental.pallas import tpu as pltpu

_BATCH = 4
_BS = 512


def _body(pe_ref, out_hbm, sems):
    i = pl.program_id(0)
    copies = [
        pltpu.make_async_copy(
            pe_ref, out_hbm.at[b, pl.ds(i * _BS, _BS), :], sems.at[b]
        )
        for b in range(_BATCH)
    ]
    for cp in copies:
        cp.start()
    for cp in copies:
        cp.wait()


def kernel(input_ids, pos_embed):
    batch, seq_len = input_ids.shape
    d = pos_embed.shape[1]
    grid = seq_len // _BS
    return pl.pallas_call(
        _body,
        grid=(grid,),
        in_specs=[pl.BlockSpec((_BS, d), lambda i: (i, 0))],
        out_specs=pl.BlockSpec(memory_space=pl.ANY),
        out_shape=jax.ShapeDtypeStruct((batch, seq_len, d), jnp.float32),
        scratch_shapes=[pltpu.SemaphoreType.DMA((_BATCH,))],
        compiler_params=pltpu.CompilerParams(
            dimension_semantics=("arbitrary",),
        ),
    )(pos_embed[:seq_len])


# speedup vs baseline: 1.2478x; 1.2478x over previous
import jax
import jax.numpy as jnp
from jax.experimental import pallas as pl
from jax.experimental.pallas import tpu as pltpu

_BATCH = 4
_BS = 1024


def _body(pe_ref, out_ref):
    out_ref[...] = jnp.broadcast_to(pe_ref[...][None, :, :], out_ref.shape)


def kernel(input_ids, pos_embed):
    batch, seq_len = input_ids.shape
    d = pos_embed.shape[1]
    grid = seq_len // _BS
    return pl.pallas_call(
        _body,
        grid=(grid,),
        in_specs=[pl.BlockSpec((_BS, d), lambda i: (i, 0))],
        out_specs=pl.BlockSpec((batch, _BS, d), lambda i: (0, i, 0)),
        out_shape=jax.ShapeDtypeStruct((batch, seq_len, d), jnp.float32),
        compiler_params=pltpu.CompilerParams(
            dimension_semantics=("arbitrary",),
        ),
    )(pos_embed[:seq_len])
